# baseline (speedup 1.0000x reference)
import jax
import jax.numpy as jnp
from jax.experimental import pallas as pl
from jax.experimental.pallas import tpu as pltpu


def _loss_kernel(idx_ref, a_ref, b_ref, len_ref, out_ref):
    ra = idx_ref[0] % 8
    rb = idx_ref[1] % 8
    a = a_ref[pl.ds(ra, 1), :]
    b = b_ref[pl.ds(rb, 1), :]
    d = jnp.abs(a - b)
    col = jax.lax.broadcasted_iota(jnp.int32, (1, 64), 1)
    d = jnp.where(col < 50, d, jnp.float32(jnp.inf))
    min_d = jnp.min(d)
    ls = len_ref[0]
    l1 = jnp.abs(min_d - ls)
    out_ref[0] = jnp.where(min_d < ls, jnp.float32(100.0) * l1, l1)


def kernel(index_vec, neighbor_index_vec, len_sum, table):
    idx = jnp.stack([index_vec[0], neighbor_index_vec[0]]).astype(jnp.int32)
    len_arr = jnp.reshape(len_sum, (1,))
    out = pl.pallas_call(
        _loss_kernel,
        grid_spec=pltpu.PrefetchScalarGridSpec(
            num_scalar_prefetch=1,
            grid=(1,),
            in_specs=[
                pl.BlockSpec((8, 64), lambda i, idx_ref: (idx_ref[0] // 8, 0)),
                pl.BlockSpec((8, 64), lambda i, idx_ref: (idx_ref[1] // 8, 0)),
                pl.BlockSpec(memory_space=pltpu.SMEM),
            ],
            out_specs=pl.BlockSpec(memory_space=pltpu.SMEM),
        ),
        out_shape=jax.ShapeDtypeStruct((1,), jnp.float32),
    )(idx, table, table, len_arr)
    return out[0]
